# speedup vs baseline: 1.4864x; 1.4864x over previous
import jax
import jax.numpy as jnp
from jax.experimental import pallas as pl


def _add_kernel(x_ref, pos_ref, out_ref):
    out_ref[...] = x_ref[...] + pos_ref[...][None]


def kernel(x, pos_table):
    batch, seq, dim = x.shape
    s_blk = 512
    n_seq = seq // s_blk
    return pl.pallas_call(
        _add_kernel,
        grid=(n_seq, batch),
        in_specs=[
            pl.BlockSpec((1, s_blk, dim), lambda i, j: (j, i, 0)),
            pl.BlockSpec((s_blk, dim), lambda i, j: (i, 0)),
        ],
        out_specs=pl.BlockSpec((1, s_blk, dim), lambda i, j: (j, i, 0)),
        out_shape=jax.ShapeDtypeStruct(x.shape, x.dtype),
    )(x, pos_table[:seq])


# speedup vs baseline: 1.6666x; 1.1212x over previous
import jax
import jax.numpy as jnp
from jax.experimental import pallas as pl


def _add_kernel(x_ref, pos_ref, out_ref):
    out_ref[...] = x_ref[...] + pos_ref[...][None]


def kernel(x, pos_table):
    batch, seq, dim = x.shape
    s_blk = 1024
    n_seq = seq // s_blk
    return pl.pallas_call(
        _add_kernel,
        grid=(n_seq, batch),
        in_specs=[
            pl.BlockSpec((1, s_blk, dim), lambda i, j: (j, i, 0)),
            pl.BlockSpec((s_blk, dim), lambda i, j: (i, 0)),
        ],
        out_specs=pl.BlockSpec((1, s_blk, dim), lambda i, j: (j, i, 0)),
        out_shape=jax.ShapeDtypeStruct(x.shape, x.dtype),
    )(x, pos_table[:seq])


# speedup vs baseline: 1.7347x; 1.0409x over previous
import jax
import jax.numpy as jnp
from jax.experimental import pallas as pl


def _add_kernel(x_ref, pos_ref, out_ref):
    out_ref[...] = x_ref[...] + pos_ref[...][None]


def kernel(x, pos_table):
    batch, seq, dim = x.shape
    s_blk = 2048
    n_seq = seq // s_blk
    return pl.pallas_call(
        _add_kernel,
        grid=(n_seq, batch),
        in_specs=[
            pl.BlockSpec((1, s_blk, dim), lambda i, j: (j, i, 0)),
            pl.BlockSpec((s_blk, dim), lambda i, j: (i, 0)),
        ],
        out_specs=pl.BlockSpec((1, s_blk, dim), lambda i, j: (j, i, 0)),
        out_shape=jax.ShapeDtypeStruct(x.shape, x.dtype),
    )(x, pos_table[:seq])
